# baseline (speedup 1.0000x reference)
import jax
import jax.numpy as jnp
from jax import lax
from jax.experimental import pallas as pl
from jax.experimental.pallas import tpu as pltpu
from jax.experimental.pallas import tpu_sc as plsc

N = 10000
E = 320000
D_IN = 128
DH = 64
K = 10

NS = 16
NC = 2
NPAD = 10240
RPT = NPAD // NS
NB = 160
NBUF = 4
EPT = E // NS
NAGG = NPAD + NS
NDEG = NAGG // NS


def _mlp_body(x_ref, w1_ref, b1_ref, w2_ref, b2_ref, o_ref):
    h1 = jnp.dot(x_ref[...], w1_ref[...], preferred_element_type=jnp.float32)
    h1 = jnp.maximum(h1 + b1_ref[...], 0.0)
    h2 = jnp.dot(h1, w2_ref[...], preferred_element_type=jnp.float32)
    h2 = h2 + b2_ref[...]
    o_ref[0] = h2[:, :DH]
    o_ref[1] = h2[:, DH:]


def _mlp(x_pad, W1, b1, W2, b2):
    R = 1024
    return pl.pallas_call(
        _mlp_body,
        grid=(NPAD // R,),
        in_specs=[
            pl.BlockSpec((R, D_IN), lambda i: (i, 0)),
            pl.BlockSpec((D_IN, D_IN), lambda i: (0, 0)),
            pl.BlockSpec((1, D_IN), lambda i: (0, 0)),
            pl.BlockSpec((D_IN, D_IN), lambda i: (0, 0)),
            pl.BlockSpec((1, D_IN), lambda i: (0, 0)),
        ],
        out_specs=pl.BlockSpec((2, R, DH), lambda i: (0, i, 0)),
        out_shape=jax.ShapeDtypeStruct((2, NPAD, DH), jnp.float32),
    )(x_pad, W1, b1.reshape(1, D_IN), W2, b2.reshape(1, D_IN))


def _rsqrt16(x):
    i = lax.bitcast_convert_type(x, jnp.int32)
    i = jnp.full((16,), 0x5F3759DF, jnp.int32) - jnp.right_shift(
        i, jnp.full((16,), 1, jnp.int32))
    y = lax.bitcast_convert_type(i, jnp.float32)
    half_x = x * 0.5
    for _ in range(3):
        y = y * (1.5 - half_x * y * y)
    return y


def _sc_body(h_hbm, src_hbm, dst_hbm, temp_hbm, out_hbm, g_hbm,
             src_v, dst_v, buf0, buf1, buf2, buf3, ones_v, temp_v,
             agg_sh, deg_sh, sem0, sem1, sem2, sem3):
    bufs = (buf0, buf1, buf2, buf3)
    sems = (sem0, sem1, sem2, sem3)
    c = lax.axis_index("c")
    s = lax.axis_index("s")
    row0 = s * RPT

    pltpu.sync_copy(src_hbm.at[c, s], src_v)
    pltpu.sync_copy(dst_hbm.at[s], dst_v)
    pltpu.sync_copy(temp_hbm, temp_v)

    ones16 = jnp.full((16,), 1.0, jnp.float32)

    def _ones_row(i, carry):
        ones_v[i] = ones16
        return carry
    lax.fori_loop(0, 128, _ones_row, 0)

    for j in range(5):
        pltpu.sync_copy(ones_v, deg_sh.at[pl.ds(s * NDEG + j * 128, 128)])
    pltpu.sync_copy(ones_v.at[pl.ds(0, 1)],
                    deg_sh.at[pl.ds(s * NDEG + 640, 1)])
    plsc.subcore_barrier()

    for b in range(NBUF):
        pltpu.async_copy(ones_v, deg_sh.at[dst_v.at[b]], sems[b], add=True)

    def _deg_blk(i, carry):
        j0 = i * NBUF
        for b in range(NBUF):
            pltpu.make_async_copy(
                ones_v, deg_sh.at[dst_v.at[j0 + b]], sems[b]).wait()
            pltpu.async_copy(ones_v, deg_sh.at[dst_v.at[j0 + b + NBUF]],
                             sems[b], add=True)
        return carry
    lax.fori_loop(0, NB // NBUF - 1, _deg_blk, 0)
    j0 = NB - NBUF
    for b in range(NBUF):
        pltpu.make_async_copy(
            ones_v, deg_sh.at[dst_v.at[j0 + b]], sems[b]).wait()
    plsc.subcore_barrier()

    for j in range(5):
        pltpu.sync_copy(deg_sh.at[pl.ds(row0 + j * 128, 128)], ones_v)

        def _dis_row(r, carry):
            ones_v[r] = _rsqrt16(ones_v[r])
            return carry
        lax.fori_loop(0, 128, _dis_row, 0)
        pltpu.sync_copy(ones_v, deg_sh.at[pl.ds(row0 + j * 128, 128)])

    tv0 = temp_v[0]
    for j in range(5):
        pltpu.sync_copy(h_hbm.at[c, pl.ds(row0 + j * 128, 128)], buf0)
        pltpu.sync_copy(deg_sh.at[pl.ds(row0 + j * 128, 128)], ones_v)

        def _init_row(r, carry):
            dsp = ones_v[r]
            for cg in range(4):
                hrow = buf0[r, pl.ds(cg * 16, 16)]
                buf1[r, pl.ds(cg * 16, 16)] = hrow * tv0
                buf2[r, pl.ds(cg * 16, 16)] = hrow * dsp
            return carry
        lax.fori_loop(0, 128, _init_row, 0)
        pltpu.sync_copy(buf1, out_hbm.at[c, pl.ds(row0 + j * 128, 128)])
        pltpu.sync_copy(buf2, g_hbm.at[pl.ds(c * NPAD + row0 + j * 128, 128)])
        pltpu.sync_copy(buf2, agg_sh.at[pl.ds(row0 + j * 128, 128)])
    plsc.subcore_barrier()

    def _gather(j, b):
        return pltpu.async_copy(g_hbm.at[src_v.at[j]], bufs[b], sems[b])

    def _round(k, carry):
        for b in range(NBUF):
            _gather(b, b)

        def _edge_blk(i, carry2):
            j0 = i * NBUF
            for b in range(NBUF):
                pltpu.make_async_copy(
                    g_hbm.at[src_v.at[j0 + b]], bufs[b], sems[b]).wait()
                pltpu.sync_copy(bufs[b], agg_sh.at[dst_v.at[j0 + b]],
                                add=True)
                _gather(j0 + b + NBUF, b)
            return carry2
        lax.fori_loop(0, NB // NBUF - 1, _edge_blk, 0)
        j0 = NB - NBUF
        for b in range(NBUF):
            pltpu.make_async_copy(
                g_hbm.at[src_v.at[j0 + b]], bufs[b], sems[b]).wait()
            pltpu.sync_copy(bufs[b], agg_sh.at[dst_v.at[j0 + b]], add=True)
        plsc.subcore_barrier()

        tk = temp_v[k + 1]
        pairs = ((buf0, buf1, sem0, sem2), (buf2, buf3, sem1, sem3))

        def _hid_ld(j):
            _, hb, sl, _ = pairs[j % 2]
            pltpu.async_copy(out_hbm.at[c, pl.ds(row0 + j * 128, 128)],
                             hb, sl)

        def _st_wait(j):
            ab, hb, _, ss = pairs[j % 2]
            pltpu.make_async_copy(
                hb, out_hbm.at[c, pl.ds(row0 + j * 128, 128)], ss).wait()
            pltpu.make_async_copy(
                ab, g_hbm.at[pl.ds(c * NPAD + row0 + j * 128, 128)],
                ss).wait()

        _hid_ld(0)
        for j in range(5):
            ab, hb, sl, ss = pairs[j % 2]
            if j + 1 <= 4:
                if j - 1 >= 0:
                    _st_wait(j - 1)
                _hid_ld(j + 1)
            pltpu.sync_copy(agg_sh.at[pl.ds(row0 + j * 128, 128)], ab)
            pltpu.sync_copy(deg_sh.at[pl.ds(row0 + j * 128, 128)], ones_v)
            pltpu.make_async_copy(
                out_hbm.at[c, pl.ds(row0 + j * 128, 128)], hb, sl).wait()

            def _upd_row(q, carry2, ab=ab, hb=hb):
                for u in range(4):
                    r = q * 4 + u
                    dsp = ones_v[r]
                    for cg in range(4):
                        hn = ab[r, pl.ds(cg * 16, 16)] * dsp
                        hb[r, pl.ds(cg * 16, 16)] = (
                            hb[r, pl.ds(cg * 16, 16)] + tk * hn)
                        ab[r, pl.ds(cg * 16, 16)] = hn * dsp
                return carry2
            lax.fori_loop(0, 32, _upd_row, 0)
            pltpu.async_copy(hb, out_hbm.at[c, pl.ds(row0 + j * 128, 128)],
                             ss)
            pltpu.async_copy(
                ab, g_hbm.at[pl.ds(c * NPAD + row0 + j * 128, 128)], ss)
            pltpu.sync_copy(ab, agg_sh.at[pl.ds(row0 + j * 128, 128)])
        _st_wait(3)
        _st_wait(4)
        plsc.subcore_barrier()
        return carry
    lax.fori_loop(0, K, _round, 0)


def _propagate(h, src_t, dst_t, tempb):
    fn = pl.kernel(
        _sc_body,
        out_type=(
            jax.ShapeDtypeStruct((NC, NPAD, DH), jnp.float32),
            jax.ShapeDtypeStruct((NC * NPAD, DH), jnp.float32),
        ),
        mesh=plsc.VectorSubcoreMesh(core_axis_name="c", subcore_axis_name="s"),
        scratch_types=[
            pltpu.VMEM((NB, 128), jnp.int32),
            pltpu.VMEM((NB, 128), jnp.int32),
            pltpu.VMEM((128, DH), jnp.float32),
            pltpu.VMEM((128, DH), jnp.float32),
            pltpu.VMEM((128, DH), jnp.float32),
            pltpu.VMEM((128, DH), jnp.float32),
            pltpu.VMEM((128, 16), jnp.float32),
            pltpu.VMEM((K + 1, 16), jnp.float32),
            pltpu.VMEM_SHARED((NAGG, DH), jnp.float32),
            pltpu.VMEM_SHARED((NAGG, 16), jnp.float32),
            pltpu.SemaphoreType.DMA,
            pltpu.SemaphoreType.DMA,
            pltpu.SemaphoreType.DMA,
            pltpu.SemaphoreType.DMA,
        ],
        compiler_params=pltpu.CompilerParams(use_tc_tiling_on_sc=False),
    )
    return fn(h, src_t, dst_t, tempb)


def kernel(x, edge_index, W1, b1, W2, b2, temp):
    x_pad = jnp.pad(x.astype(jnp.float32), ((0, NPAD - N), (0, 0)))
    h = _mlp(x_pad, W1, b1, W2, b2)

    src = edge_index[0].astype(jnp.int32).reshape(NS, EPT)
    dst = edge_index[1].astype(jnp.int32).reshape(NS, EPT)
    npad_e = NB * 128 - EPT
    tid = jnp.arange(NS, dtype=jnp.int32)[:, None]
    ppos = jnp.arange(npad_e, dtype=jnp.int32)[None, :]
    pad_src = (ppos * 997 + tid * RPT) % NPAD
    pad_dst = jnp.broadcast_to(NPAD + tid, (NS, npad_e))
    src_flat = jnp.concatenate([src, pad_src], axis=1).reshape(NS, NB, 128)
    src_t = src_flat[None] + (
        NPAD * jnp.arange(NC, dtype=jnp.int32))[:, None, None, None]
    dst_t = jnp.concatenate([dst, pad_dst], axis=1).reshape(NS, NB, 128)

    tempb = jnp.broadcast_to(temp.astype(jnp.float32)[:, None], (K + 1, 16))

    out, _ = _propagate(h, src_t, dst_t, tempb)
    return jnp.concatenate([out[0, :N], out[1, :N]], axis=1)


# speedup vs baseline: 1.0014x; 1.0014x over previous
import jax
import jax.numpy as jnp
from jax import lax
from jax.experimental import pallas as pl
from jax.experimental.pallas import tpu as pltpu
from jax.experimental.pallas import tpu_sc as plsc

N = 10000
E = 320000
D_IN = 128
DH = 64
K = 10

NS = 16
NC = 2
NPAD = 10240
RPT = NPAD // NS
NB = 160
NBUF = 4
EPT = E // NS
NAGG = NPAD + NS
NDEG = NAGG // NS


def _mlp_body(x_ref, w1_ref, b1_ref, w2_ref, b2_ref, o_ref):
    h1 = jnp.dot(x_ref[...], w1_ref[...], preferred_element_type=jnp.float32)
    h1 = jnp.maximum(h1 + b1_ref[...], 0.0)
    h2 = jnp.dot(h1, w2_ref[...], preferred_element_type=jnp.float32)
    h2 = h2 + b2_ref[...]
    o_ref[0] = h2[:, :DH]
    o_ref[1] = h2[:, DH:]


def _mlp(x_pad, W1, b1, W2, b2):
    R = 1024
    return pl.pallas_call(
        _mlp_body,
        grid=(NPAD // R,),
        in_specs=[
            pl.BlockSpec((R, D_IN), lambda i: (i, 0)),
            pl.BlockSpec((D_IN, D_IN), lambda i: (0, 0)),
            pl.BlockSpec((1, D_IN), lambda i: (0, 0)),
            pl.BlockSpec((D_IN, D_IN), lambda i: (0, 0)),
            pl.BlockSpec((1, D_IN), lambda i: (0, 0)),
        ],
        out_specs=pl.BlockSpec((2, R, DH), lambda i: (0, i, 0)),
        out_shape=jax.ShapeDtypeStruct((2, NPAD, DH), jnp.float32),
    )(x_pad, W1, b1.reshape(1, D_IN), W2, b2.reshape(1, D_IN))


def _rsqrt16(x):
    i = lax.bitcast_convert_type(x, jnp.int32)
    i = jnp.full((16,), 0x5F3759DF, jnp.int32) - jnp.right_shift(
        i, jnp.full((16,), 1, jnp.int32))
    y = lax.bitcast_convert_type(i, jnp.float32)
    half_x = x * 0.5
    for _ in range(3):
        y = y * (1.5 - half_x * y * y)
    return y


def _sc_body(h_hbm, src_hbm, dst_hbm, temp_hbm, out_hbm, g_hbm,
             src_v, dst_v, buf0, buf1, buf2, buf3, ones_v, temp_v,
             agg_sh, deg_sh, sem0, sem1, sem2, sem3):
    bufs = (buf0, buf1, buf2, buf3)
    sems = (sem0, sem1, sem2, sem3)
    c = lax.axis_index("c")
    s = lax.axis_index("s")
    row0 = s * RPT

    pltpu.sync_copy(src_hbm.at[c, s], src_v)
    pltpu.sync_copy(dst_hbm.at[s], dst_v)
    pltpu.sync_copy(temp_hbm, temp_v)

    ones16 = jnp.full((16,), 1.0, jnp.float32)

    def _ones_row(i, carry):
        ones_v[i] = ones16
        return carry
    lax.fori_loop(0, 128, _ones_row, 0)

    for j in range(5):
        pltpu.sync_copy(ones_v, deg_sh.at[pl.ds(s * NDEG + j * 128, 128)])
    pltpu.sync_copy(ones_v.at[pl.ds(0, 1)],
                    deg_sh.at[pl.ds(s * NDEG + 640, 1)])
    plsc.subcore_barrier()

    for b in range(NBUF):
        pltpu.async_copy(ones_v, deg_sh.at[dst_v.at[b]], sems[b], add=True)

    def _deg_blk(i, carry):
        j0 = i * NBUF
        for b in range(NBUF):
            pltpu.make_async_copy(
                ones_v, deg_sh.at[dst_v.at[j0 + b]], sems[b]).wait()
            pltpu.async_copy(ones_v, deg_sh.at[dst_v.at[j0 + b + NBUF]],
                             sems[b], add=True)
        return carry
    lax.fori_loop(0, NB // NBUF - 1, _deg_blk, 0)
    j0 = NB - NBUF
    for b in range(NBUF):
        pltpu.make_async_copy(
            ones_v, deg_sh.at[dst_v.at[j0 + b]], sems[b]).wait()
    plsc.subcore_barrier()

    for j in range(5):
        pltpu.sync_copy(deg_sh.at[pl.ds(row0 + j * 128, 128)], ones_v)

        def _dis_row(r, carry):
            ones_v[r] = _rsqrt16(ones_v[r])
            return carry
        lax.fori_loop(0, 128, _dis_row, 0)
        pltpu.sync_copy(ones_v, deg_sh.at[pl.ds(row0 + j * 128, 128)])

    tv0 = temp_v[0]
    for j in range(5):
        pltpu.sync_copy(h_hbm.at[c, pl.ds(row0 + j * 128, 128)], buf0)
        pltpu.sync_copy(deg_sh.at[pl.ds(row0 + j * 128, 128)], ones_v)

        def _init_row(r, carry):
            dsp = ones_v[r]
            for cg in range(4):
                hrow = buf0[r, pl.ds(cg * 16, 16)]
                buf1[r, pl.ds(cg * 16, 16)] = hrow * tv0
                buf2[r, pl.ds(cg * 16, 16)] = hrow * dsp
            return carry
        lax.fori_loop(0, 128, _init_row, 0)
        pltpu.sync_copy(buf1, out_hbm.at[c, pl.ds(row0 + j * 128, 128)])
        pltpu.sync_copy(buf2, g_hbm.at[pl.ds(c * NPAD + row0 + j * 128, 128)])
        pltpu.sync_copy(buf2, agg_sh.at[pl.ds(row0 + j * 128, 128)])
    plsc.subcore_barrier()

    def _gather(j, b):
        return pltpu.async_copy(g_hbm.at[src_v.at[j]], bufs[b], sems[b])

    def _round(k, carry):
        for b in range(NBUF):
            _gather(b, b)

        def _edge_blk(i, carry2):
            j0 = i * NBUF
            for b in range(NBUF):
                pltpu.make_async_copy(
                    g_hbm.at[src_v.at[j0 + b]], bufs[b], sems[b]).wait()
                pltpu.sync_copy(bufs[b], agg_sh.at[dst_v.at[j0 + b]],
                                add=True)
                _gather(j0 + b + NBUF, b)
            return carry2
        lax.fori_loop(0, NB // NBUF - 1, _edge_blk, 0)
        j0 = NB - NBUF
        for b in range(NBUF):
            pltpu.make_async_copy(
                g_hbm.at[src_v.at[j0 + b]], bufs[b], sems[b]).wait()
            pltpu.sync_copy(bufs[b], agg_sh.at[dst_v.at[j0 + b]], add=True)
        plsc.subcore_barrier()

        tk = temp_v[k + 1]
        pairs = ((buf0, buf1, sem0, sem2), (buf2, buf3, sem1, sem3))

        def _hid_ld(j):
            _, hb, sl, _ = pairs[j % 2]
            pltpu.async_copy(out_hbm.at[c, pl.ds(row0 + j * 128, 128)],
                             hb, sl)

        def _st_wait(j):
            ab, hb, _, ss = pairs[j % 2]
            pltpu.make_async_copy(
                hb, out_hbm.at[c, pl.ds(row0 + j * 128, 128)], ss).wait()
            pltpu.make_async_copy(
                ab, g_hbm.at[pl.ds(c * NPAD + row0 + j * 128, 128)],
                ss).wait()

        _hid_ld(0)
        for j in range(5):
            ab, hb, sl, ss = pairs[j % 2]
            if j + 1 <= 4:
                if j - 1 >= 0:
                    _st_wait(j - 1)
                _hid_ld(j + 1)
            pltpu.sync_copy(agg_sh.at[pl.ds(row0 + j * 128, 128)], ab)
            pltpu.sync_copy(deg_sh.at[pl.ds(row0 + j * 128, 128)], ones_v)
            pltpu.make_async_copy(
                out_hbm.at[c, pl.ds(row0 + j * 128, 128)], hb, sl).wait()

            def _upd_row(r, carry2, ab=ab, hb=hb):
                dsp = ones_v[r]
                for cg in range(4):
                    hn = ab[r, pl.ds(cg * 16, 16)] * dsp
                    hb[r, pl.ds(cg * 16, 16)] = (
                        hb[r, pl.ds(cg * 16, 16)] + tk * hn)
                    ab[r, pl.ds(cg * 16, 16)] = hn * dsp
                return carry2
            lax.fori_loop(0, 128, _upd_row, 0)
            pltpu.async_copy(hb, out_hbm.at[c, pl.ds(row0 + j * 128, 128)],
                             ss)
            pltpu.async_copy(
                ab, g_hbm.at[pl.ds(c * NPAD + row0 + j * 128, 128)], ss)
            pltpu.sync_copy(ab, agg_sh.at[pl.ds(row0 + j * 128, 128)])
        _st_wait(3)
        _st_wait(4)
        plsc.subcore_barrier()
        return carry
    lax.fori_loop(0, K, _round, 0)


def _propagate(h, src_t, dst_t, tempb):
    fn = pl.kernel(
        _sc_body,
        out_type=(
            jax.ShapeDtypeStruct((NC, NPAD, DH), jnp.float32),
            jax.ShapeDtypeStruct((NC * NPAD, DH), jnp.float32),
        ),
        mesh=plsc.VectorSubcoreMesh(core_axis_name="c", subcore_axis_name="s"),
        scratch_types=[
            pltpu.VMEM((NB, 128), jnp.int32),
            pltpu.VMEM((NB, 128), jnp.int32),
            pltpu.VMEM((128, DH), jnp.float32),
            pltpu.VMEM((128, DH), jnp.float32),
            pltpu.VMEM((128, DH), jnp.float32),
            pltpu.VMEM((128, DH), jnp.float32),
            pltpu.VMEM((128, 16), jnp.float32),
            pltpu.VMEM((K + 1, 16), jnp.float32),
            pltpu.VMEM_SHARED((NAGG, DH), jnp.float32),
            pltpu.VMEM_SHARED((NAGG, 16), jnp.float32),
            pltpu.SemaphoreType.DMA,
            pltpu.SemaphoreType.DMA,
            pltpu.SemaphoreType.DMA,
            pltpu.SemaphoreType.DMA,
        ],
        compiler_params=pltpu.CompilerParams(use_tc_tiling_on_sc=False),
    )
    return fn(h, src_t, dst_t, tempb)


def kernel(x, edge_index, W1, b1, W2, b2, temp):
    x_pad = jnp.pad(x.astype(jnp.float32), ((0, NPAD - N), (0, 0)))
    h = _mlp(x_pad, W1, b1, W2, b2)

    src = edge_index[0].astype(jnp.int32).reshape(NS, EPT)
    dst = edge_index[1].astype(jnp.int32).reshape(NS, EPT)
    npad_e = NB * 128 - EPT
    tid = jnp.arange(NS, dtype=jnp.int32)[:, None]
    ppos = jnp.arange(npad_e, dtype=jnp.int32)[None, :]
    pad_src = (ppos * 997 + tid * RPT) % NPAD
    pad_dst = jnp.broadcast_to(NPAD + tid, (NS, npad_e))
    src_flat = jnp.concatenate([src, pad_src], axis=1).reshape(NS, NB, 128)
    src_t = src_flat[None] + (
        NPAD * jnp.arange(NC, dtype=jnp.int32))[:, None, None, None]
    dst_t = jnp.concatenate([dst, pad_dst], axis=1).reshape(NS, NB, 128)

    tempb = jnp.broadcast_to(temp.astype(jnp.float32)[:, None], (K + 1, 16))

    out, _ = _propagate(h, src_t, dst_t, tempb)
    return jnp.concatenate([out[0, :N], out[1, :N]], axis=1)
